# baseline (speedup 1.0000x reference)
import functools
import math

import jax
import jax.numpy as jnp
from jax import lax
from jax.experimental import pallas as pl
from jax.experimental.pallas import tpu as pltpu
from jax.experimental.pallas import tpu_sc as plsc

D_MODEL = 64
SCALE = math.sqrt(D_MODEL)


def _sc_body(x_hbm, p_hbm, out_hbm, xbuf, pbuf, sem_in, sem_p, sem_out):
    nworkers = 32
    rows_per_w = 409600 // nworkers
    chunk = 256
    nchunks = rows_per_w // chunk
    wid = lax.axis_index("s") * 2 + lax.axis_index("c")
    wstart = wid * rows_per_w

    def body(g, _):
        base = wstart + g * chunk
        qbase = (base // 2048) * 64 + ((base // 256) % 8) * 8
        pltpu.sync_copy(x_hbm.at[pl.ds(base, chunk)], xbuf)
        pltpu.sync_copy(p_hbm.at[pl.ds(qbase, 8)], pbuf)

        def inner(bt, _):
            for di in range(8):
                r = bt * 8 + di
                for j in range(8):
                    sl = pl.ds(j * 16, 16)
                    xbuf[r, sl] = xbuf[r, sl] * SCALE + pbuf[di, sl]
            return 0

        lax.fori_loop(0, 32, inner, 0)
        pltpu.sync_copy(xbuf, out_hbm.at[pl.ds(base, chunk)])
        return 0

    lax.fori_loop(0, nchunks, body, 0)


def kernel(x, pos_emb):
    B, L, D = x.shape
    xt = x.transpose(1, 2, 0)
    x5 = xt.reshape(L, D // 8, 8, B // 128, 128)
    xlin = x5.transpose(0, 1, 3, 2, 4).reshape(L * D * B // 128, 128)
    pbig = jnp.broadcast_to(pos_emb.reshape(L * D, 1), (L * D, 128))

    mesh = plsc.VectorSubcoreMesh(core_axis_name="c", subcore_axis_name="s")
    f = functools.partial(
        pl.kernel,
        mesh=mesh,
        out_type=jax.ShapeDtypeStruct((L * D * B // 128, 128), jnp.float32),
        scratch_types=[
            pltpu.VMEM((256, 128), jnp.float32),
            pltpu.VMEM((8, 128), jnp.float32),
            pltpu.SemaphoreType.DMA,
            pltpu.SemaphoreType.DMA,
            pltpu.SemaphoreType.DMA,
        ],
    )(_sc_body)
    outlin = f(xlin, pbig)
    o5 = outlin.reshape(L, D // 8, B // 128, 8, 128).transpose(0, 1, 3, 2, 4)
    return o5.reshape(L, D, B).transpose(2, 0, 1)


# speedup vs baseline: 1.2304x; 1.2304x over previous
import functools
import math

import jax
import jax.numpy as jnp
from jax import lax
from jax.experimental import pallas as pl
from jax.experimental.pallas import tpu as pltpu
from jax.experimental.pallas import tpu_sc as plsc

D_MODEL = 64
SCALE = math.sqrt(D_MODEL)

NW = 32
CHUNK = 256
ROWS = 409600
ROWS_PER_W = ROWS // NW
NCHUNK = ROWS_PER_W // CHUNK


def _sc_body(x_hbm, p_hbm, out_hbm, xb0, xb1, pb0, pb1, sin, sp, sout):
    wid = lax.axis_index("s") * 2 + lax.axis_index("c")
    wstart = wid * ROWS_PER_W
    xbufs = (xb0, xb1)
    pbufs = (pb0, pb1)

    def qbase_of(base):
        return (base // 2048) * 64 + ((base // 256) % 8) * 8

    def start_in(g, b):
        base = wstart + g * CHUNK
        pltpu.make_async_copy(
            x_hbm.at[pl.ds(base, CHUNK)], xbufs[b], sin.at[b]).start()
        pltpu.make_async_copy(
            p_hbm.at[pl.ds(qbase_of(base), 8)], pbufs[b], sp.at[b]).start()

    def wait_in(g, b):
        base = g * CHUNK
        pltpu.make_async_copy(
            x_hbm.at[pl.ds(base, CHUNK)], xbufs[b], sin.at[b]).wait()
        pltpu.make_async_copy(
            p_hbm.at[pl.ds(qbase_of(base), 8)], pbufs[b], sp.at[b]).wait()

    def start_out(g, b):
        base = wstart + g * CHUNK
        pltpu.make_async_copy(
            xbufs[b], out_hbm.at[pl.ds(base, CHUNK)], sout.at[b]).start()

    def wait_out(g, b):
        base = g * CHUNK
        pltpu.make_async_copy(
            xbufs[b], out_hbm.at[pl.ds(base, CHUNK)], sout.at[b]).wait()

    def compute(b):
        xb, pb = xbufs[b], pbufs[b]

        def inner(bt, _):
            for di in range(8):
                r = bt * 8 + di
                for j in range(8):
                    sl = pl.ds(j * 16, 16)
                    xb[r, sl] = xb[r, sl] * SCALE + pb[di, sl]
            return 0

        lax.fori_loop(0, 32, inner, 0)

    start_in(0, 0)

    def outer(k, _):
        for b in (0, 1):
            g = 2 * k + b

            @pl.when(jnp.logical_and(g >= 1, g + 1 < NCHUNK))
            def _():
                wait_out(g - 1, 1 - b)

            @pl.when(g + 1 < NCHUNK)
            def _():
                start_in(g + 1, 1 - b)

            wait_in(g, b)
            compute(b)
            start_out(g, b)
        return 0

    lax.fori_loop(0, NCHUNK // 2, outer, 0)
    wait_out(NCHUNK - 2, 0)
    wait_out(NCHUNK - 1, 1)


def kernel(x, pos_emb):
    B, L, D = x.shape
    xt = x.transpose(1, 2, 0)
    x5 = xt.reshape(L, D // 8, 8, B // 128, 128)
    xlin = x5.transpose(0, 1, 3, 2, 4).reshape(ROWS, 128)
    pbig = jnp.broadcast_to(pos_emb.reshape(L * D, 1), (L * D, 128))

    mesh = plsc.VectorSubcoreMesh(core_axis_name="c", subcore_axis_name="s")
    f = functools.partial(
        pl.kernel,
        mesh=mesh,
        out_type=jax.ShapeDtypeStruct((ROWS, 128), jnp.float32),
        scratch_types=[
            pltpu.VMEM((CHUNK, 128), jnp.float32),
            pltpu.VMEM((CHUNK, 128), jnp.float32),
            pltpu.VMEM((8, 128), jnp.float32),
            pltpu.VMEM((8, 128), jnp.float32),
            pltpu.SemaphoreType.DMA((2,)),
            pltpu.SemaphoreType.DMA((2,)),
            pltpu.SemaphoreType.DMA((2,)),
        ],
    )(_sc_body)
    outlin = f(xlin, pbig)
    o5 = outlin.reshape(L, D // 8, B // 128, 8, 128).transpose(0, 1, 3, 2, 4)
    return o5.reshape(L, D, B).transpose(2, 0, 1)


# speedup vs baseline: 5.3529x; 4.3504x over previous
import functools
import math

import jax
import jax.numpy as jnp
from jax.experimental import pallas as pl
from jax.experimental.pallas import tpu as pltpu


def _fma_kernel(p_nat_ref, x_ref, o_ref, p_scr, *, scale, lblk):
    @pl.when(pl.program_id(0) == 0)
    def _():
        p_scr[...] = jnp.transpose(p_nat_ref[...], (1, 0))

    i = pl.program_id(0)
    p = p_scr[pl.ds(i * lblk, lblk), :]
    o_ref[...] = x_ref[...] * scale + p[:, :, None]


def kernel(x, pos_emb):
    B, L, D = x.shape
    scale = math.sqrt(D)
    xt = x.transpose(1, 2, 0)
    p_nat = pos_emb.transpose(1, 0)
    lblk = 8
    out = pl.pallas_call(
        functools.partial(_fma_kernel, scale=scale, lblk=lblk),
        grid=(L // lblk,),
        in_specs=[
            pl.BlockSpec((D, L), lambda i: (0, 0)),
            pl.BlockSpec((lblk, D, B), lambda i: (i, 0, 0)),
        ],
        out_specs=pl.BlockSpec((lblk, D, B), lambda i: (i, 0, 0)),
        out_shape=jax.ShapeDtypeStruct((L, D, B), x.dtype),
        scratch_shapes=[pltpu.VMEM((L, D), jnp.float32)],
    )(p_nat, xt)
    return out.transpose(2, 0, 1)
